# Initial kernel scaffold; baseline (speedup 1.0000x reference)
import jax
import jax.numpy as jnp
from jax.experimental import pallas as pl


def kernel(hidden_states, norm_weight, router_weight, expert_bias, sw1, sw2, sw3, ew1, ew2, ew3):
    raise NotImplementedError("write your pallas kernel here")


# speedup vs baseline: 1.0147x; 1.0147x over previous
import functools

import jax
import jax.numpy as jnp
from jax.experimental import pallas as pl
from jax.experimental.pallas import tpu as pltpu

S, D, F, E, K = 2048, 768, 1536, 8, 2
EPS = 1e-06
BALANCE_FACTOR = 1e-4
LANES = 128


def _router_body(x_ref, nw_ref, wr_ref, bias_ref, hb_ref, mask_ref, loss_ref):
    x = x_ref[...]
    ms = jnp.mean(x * x, axis=1, keepdims=True)
    h = x * jax.lax.rsqrt(ms + EPS) * nw_ref[...]
    logits = jnp.dot(h, wr_ref[...], preferred_element_type=jnp.float32)
    logits = logits + bias_ref[...]
    m = jnp.max(logits, axis=1, keepdims=True)
    p = jnp.exp(logits - m)
    probs = p / jnp.sum(p, axis=1, keepdims=True)
    lane = jax.lax.broadcasted_iota(jnp.int32, probs.shape, 1)
    p1 = jnp.max(probs, axis=1, keepdims=True)
    i1 = jnp.min(jnp.where(probs == p1, lane, LANES - 1), axis=1, keepdims=True)
    probs2 = jnp.where(lane == i1, -1.0, probs)
    p2 = jnp.max(probs2, axis=1, keepdims=True)
    i2 = jnp.min(jnp.where(probs2 == p2, lane, LANES - 1), axis=1, keepdims=True)
    s = p1 + p2
    mask = jnp.where(lane == i1, p1 / s, 0.0) + jnp.where(lane == i2, p2 / s, 0.0)
    mask = mask + jnp.where(lane == E, 1.0, 0.0)
    mask_ref[...] = mask[:, :16]
    hb_ref[...] = h.astype(jnp.bfloat16)
    load = jnp.sum(probs, axis=0, keepdims=True) / S
    tl = 1.0 / E
    ll = jnp.where(lane[:1, :] < E,
                   tl * (jnp.log(tl) - jnp.log(jnp.maximum(load, 1e-30))), 0.0)
    loss_ref[...] = jnp.sum(ll, axis=1, keepdims=True) / E * BALANCE_FACTOR


def _expert_body(hb_ref, res_ref, mask_ref, w1_ref, w3_ref, w2_ref, out_ref):
    e = pl.program_id(0)
    xb = hb_ref[...]
    w1 = w1_ref[0].astype(jnp.bfloat16)
    w3 = w3_ref[0].astype(jnp.bfloat16)
    w2 = w2_ref[0].astype(jnp.bfloat16)
    nt = (((1,), (1,)), ((), ()))
    a = jax.lax.dot_general(xb, w1, nt, preferred_element_type=jnp.float32)
    b = jax.lax.dot_general(xb, w3, nt, preferred_element_type=jnp.float32)
    g = (a * (1.0 / (1.0 + jnp.exp(-a))) * b).astype(jnp.bfloat16)
    y = jax.lax.dot_general(g, w2, nt, preferred_element_type=jnp.float32)
    lane16 = jax.lax.broadcasted_iota(jnp.int32, (S, 16), 1)
    mcol = jnp.sum(mask_ref[...] * (lane16 == e), axis=1, keepdims=True)
    contrib = y * mcol

    @pl.when(e == 0)
    def _():
        out_ref[...] = res_ref[...] + contrib

    @pl.when(e > 0)
    def _():
        out_ref[...] += contrib


@jax.jit
def kernel(hidden_states, norm_weight, router_weight, expert_bias,
           sw1, sw2, sw3, ew1, ew2, ew3):
    x = hidden_states.reshape(S, D)
    nw = norm_weight.reshape(1, D)
    wr = jnp.zeros((D, LANES), jnp.float32).at[:, :E].set(router_weight.T)
    bias = jnp.full((1, LANES), -1e30, jnp.float32).at[0, :E].set(expert_bias)

    hb, mask16, loss = pl.pallas_call(
        _router_body,
        out_shape=(
            jax.ShapeDtypeStruct((S, D), jnp.bfloat16),
            jax.ShapeDtypeStruct((S, 16), jnp.float32),
            jax.ShapeDtypeStruct((1, 1), jnp.float32),
        ),
    )(x, nw, wr, bias)

    w1 = jnp.concatenate([ew1, sw1[None]], axis=0)
    w3 = jnp.concatenate([ew3, sw3[None]], axis=0)
    w2 = jnp.concatenate([ew2, sw2[None]], axis=0)

    out = pl.pallas_call(
        _expert_body,
        grid=(E + 1,),
        in_specs=[
            pl.BlockSpec((S, D), lambda e: (0, 0)),
            pl.BlockSpec((S, D), lambda e: (0, 0)),
            pl.BlockSpec((S, 16), lambda e: (0, 0)),
            pl.BlockSpec((1, F, D), lambda e: (e, 0, 0)),
            pl.BlockSpec((1, F, D), lambda e: (e, 0, 0)),
            pl.BlockSpec((1, D, F), lambda e: (e, 0, 0)),
        ],
        out_specs=pl.BlockSpec((S, D), lambda e: (0, 0)),
        out_shape=jax.ShapeDtypeStruct((S, D), jnp.float32),
    )(hb, x, mask16, w1, w3, w2)

    return out.reshape(1, S, D), loss.reshape(())
